# baseline (speedup 1.0000x reference)
import functools

import jax
import jax.numpy as jnp
from jax import lax
from jax.experimental import pallas as pl
from jax.experimental.pallas import tpu as pltpu
from jax.experimental.pallas import tpu_sc as plsc

N = 10000
E = 320000
B = 8
NPER = 1250
F = 128
K1 = 625
K2 = 313
NPAD = 10240
ZROW = N
NC = 2
NS = 16
NTILES = NC * NS
CH = 128
NSTEP = 79
EPT = NSTEP * CH
EPAD = NTILES * EPT
RPS = NPAD // NS
FH = 64
KB = 4
NSTEP2 = 160
EPAD2 = NS * NSTEP2 * CH
CR = 128
NPERP = 1280

@functools.cache
def _mesh():
    return plsc.VectorSubcoreMesh(
        core_axis_name="c", subcore_axis_name="s",
        num_cores=NC, num_subcores=NS)


def _maskedge_body(m_hbm, src_hbm, dst_hbm, ones_hbm, zdeg_hbm,
                   idxg_hbm, idxs_hbm, degp_hbm,
                   m_v, s_v, d_v, og_v, os_v, isb_v, ones_v, deg_sp):
    c = lax.axis_index("c")
    s = lax.axis_index("s")
    wid = c * NS + s
    pltpu.sync_copy(m_hbm, m_v)
    pltpu.sync_copy(src_hbm.at[wid], s_v)
    pltpu.sync_copy(dst_hbm.at[wid], d_v)
    pltpu.sync_copy(ones_hbm, ones_v)
    pltpu.sync_copy(zdeg_hbm.at[pl.ds(s * RPS, RPS)],
                    deg_sp.at[pl.ds(s * RPS, RPS)])
    plsc.subcore_barrier()

    def step(j, carry):
        for k in range(CH // 16):
            s16 = s_v[j, pl.ds(k * 16, 16)]
            d16 = d_v[j, pl.ds(k * 16, 16)]
            mg = plsc.load_gather(m_v, [s16])
            md = plsc.load_gather(m_v, [d16])
            keep = (mg * md) > 0.5
            trash16 = lax.iota(jnp.int32, 16) + (ZROW + 16 * k)
            os16 = jnp.where(keep, d16, trash16)
            og_v[j, pl.ds(k * 16, 16)] = jnp.where(keep, s16, trash16)
            os_v[j, pl.ds(k * 16, 16)] = os16
            isb_v[pl.ds(k * 16, 16)] = os16
        pltpu.sync_copy(ones_v, deg_sp.at[isb_v], add=True)
        return carry

    lax.fori_loop(0, NSTEP, step, 0)
    pltpu.sync_copy(og_v, idxg_hbm.at[wid])
    pltpu.sync_copy(os_v, idxs_hbm.at[wid])
    plsc.subcore_barrier()
    pltpu.sync_copy(deg_sp.at[pl.ds(s * RPS, RPS)],
                    degp_hbm.at[c, pl.ds(s * RPS, RPS)])


@functools.cache
def _maskedge_kernel():
    return pl.kernel(
        _maskedge_body,
        out_type=(
            jax.ShapeDtypeStruct((NTILES, NSTEP, CH), jnp.int32),
            jax.ShapeDtypeStruct((NTILES, NSTEP, CH), jnp.int32),
            jax.ShapeDtypeStruct((NC, NPAD, FH), jnp.float32),
        ),
        mesh=_mesh(),
        compiler_params=pltpu.CompilerParams(
            needs_layout_passes=False, use_tc_tiling_on_sc=False),
        scratch_types=[
            pltpu.VMEM((NPAD,), jnp.float32),
            pltpu.VMEM((NSTEP, CH), jnp.int32),
            pltpu.VMEM((NSTEP, CH), jnp.int32),
            pltpu.VMEM((NSTEP, CH), jnp.int32),
            pltpu.VMEM((NSTEP, CH), jnp.int32),
            pltpu.VMEM((CH,), jnp.int32),
            pltpu.VMEM((CH, FH), jnp.float32),
            pltpu.VMEM_SHARED((NPAD, FH), jnp.float32),
        ],
    )


def _maskedge(m, src3, dst3, ones_cd, zdeg):
    return _maskedge_kernel()(m, src3, dst3, ones_cd, zdeg)


def _mega_body(info, table_hbm, idxg_hbm, idxs_hbm, dinvb_hbm, self_hbm,
               zr_hbm, *refs):
    if info:
        (h_hbm, nei_hbm, igb_v, isb_v, rows_v,
         sb_v, db_v, bb_v, hb_v, acc_sp, isem, gsem, ssem) = refs
    else:
        (h_hbm, igb_v, isb_v, rows_v,
         sb_v, db_v, bb_v, hb_v, acc_sp, isem, gsem, ssem) = refs
    c = lax.axis_index("c")
    s = lax.axis_index("s")
    pltpu.sync_copy(zr_hbm.at[pl.ds(s * RPS, RPS)],
                    acc_sp.at[pl.ds(s * RPS, RPS)])
    plsc.subcore_barrier()

    def scat(src_hbm):
        def step(g, carry):
            base = g * KB
            xd = []
            for b in range(KB):
                j = base + b
                xd.append((pltpu.async_copy(idxg_hbm.at[s, j], igb_v[b], isem),
                           pltpu.async_copy(idxs_hbm.at[s, j], isb_v[b], isem)))
            gd = []
            for b in range(KB):
                xd[b][0].wait()
                xd[b][1].wait()
                gd.append(pltpu.async_copy(src_hbm.at[c].at[igb_v[b]],
                                           rows_v[b], gsem))
            sd = []
            for b in range(KB):
                gd[b].wait()
                sd.append(pltpu.async_copy(rows_v[b], acc_sp.at[isb_v[b]],
                                           ssem, add=True))
            for b in range(KB):
                sd[b].wait()
            return carry
        lax.fori_loop(0, NSTEP2 // KB, step, 0)

    scat(table_hbm)
    plsc.subcore_barrier()

    for q in range(RPS // CR):
        row0 = s * RPS + q * CR
        pltpu.sync_copy(acc_sp.at[pl.ds(row0, CR)], sb_v)
        pltpu.sync_copy(dinvb_hbm.at[pl.ds(row0, CR)], db_v)
        pltpu.sync_copy(self_hbm.at[c, pl.ds(row0, CR)], bb_v)

        def erow(r, carry):
            for k in range(FH // 16):
                sl = pl.ds(k * 16, 16)
                hb_v[r, sl] = jnp.maximum(
                    db_v[r, sl] * sb_v[r, sl] + bb_v[r, sl], 0.0)
            return carry
        lax.fori_loop(0, CR, erow, 0)
        pltpu.sync_copy(hb_v, h_hbm.at[c, pl.ds(row0, CR)])
    pltpu.sync_copy(zr_hbm.at[pl.ds(s * RPS, RPS)],
                    acc_sp.at[pl.ds(s * RPS, RPS)])
    plsc.subcore_barrier()

    if info:
        scat(h_hbm)
        plsc.subcore_barrier()
        pltpu.sync_copy(acc_sp.at[pl.ds(s * RPS, RPS)],
                        nei_hbm.at[c, pl.ds(s * RPS, RPS)])


@functools.cache
def _mega_kernel(info):
    outs = [jax.ShapeDtypeStruct((NC, NPAD, FH), jnp.float32)]
    if info:
        outs.append(jax.ShapeDtypeStruct((NC, NPAD, FH), jnp.float32))
    return pl.kernel(
        functools.partial(_mega_body, info),
        out_type=tuple(outs),
        mesh=_mesh(),
        compiler_params=pltpu.CompilerParams(
            needs_layout_passes=False, use_tc_tiling_on_sc=False),
        scratch_types=[
            [pltpu.VMEM((CH,), jnp.int32) for _ in range(KB)],
            [pltpu.VMEM((CH,), jnp.int32) for _ in range(KB)],
            [pltpu.VMEM((CH, FH), jnp.float32) for _ in range(KB)],
            pltpu.VMEM((CR, FH), jnp.float32),
            pltpu.VMEM((CR, FH), jnp.float32),
            pltpu.VMEM((CR, FH), jnp.float32),
            pltpu.VMEM((CR, FH), jnp.float32),
            pltpu.VMEM_SHARED((NPAD, FH), jnp.float32),
            pltpu.SemaphoreType.DMA,
            pltpu.SemaphoreType.DMA,
            pltpu.SemaphoreType.DMA,
        ],
    )


def _mega(table2, idxg, idxs, dinvb, selfh, zhalf, info):
    def pad2(a):
        flat = a.reshape(-1)
        spread = ZROW + jnp.arange(EPAD2 - EPAD, dtype=jnp.int32) % 128
        flat = jnp.concatenate([flat, spread])
        return flat.reshape(NS, NSTEP2, CH)
    return _mega_kernel(info)(table2, pad2(idxg), pad2(idxs),
                              dinvb, selfh, zhalf)


BM = 1024


def _mm_body(half_in, x_ref, m_ref, degp_ref, w_ref, b_ref,
             tab_ref, dinvb_ref, self_ref):
    i = pl.program_id(0)
    dp = degp_ref[...]
    deg = dp[0, :, 0:1] + dp[1, :, 0:1]
    dinv = lax.rsqrt(deg + 1.0)
    rows = i * BM + lax.broadcasted_iota(jnp.int32, (BM, 1), 0)
    valid = (rows < N).astype(jnp.float32)
    if half_in:
        xv = x_ref[...]
        x = jnp.concatenate([xv[0], xv[1]], axis=1)
    else:
        x = x_ref[...]
    xm = x * m_ref[...]
    h = jnp.dot(xm, w_ref[...], preferred_element_type=jnp.float32)
    hs = h * dinv
    tab_ref[0, :, :] = hs[:, 0:FH]
    tab_ref[1, :, :] = hs[:, FH:F]
    dinvb_ref[...] = jnp.broadcast_to(dinv * valid, (BM, FH))
    selfv = (dinv * dinv * h + b_ref[...]) * valid
    self_ref[0, :, :] = selfv[:, 0:FH]
    self_ref[1, :, :] = selfv[:, FH:F]


def _tc_mm(x, m, degp, w, b, half_in):
    xspec = (pl.BlockSpec((NC, BM, FH), lambda i: (0, i, 0)) if half_in
             else pl.BlockSpec((BM, F), lambda i: (i, 0)))
    return pl.pallas_call(
        functools.partial(_mm_body, half_in),
        grid=(NPAD // BM,),
        in_specs=[
            xspec,
            pl.BlockSpec((BM, 1), lambda i: (i, 0)),
            pl.BlockSpec((NC, BM, FH), lambda i: (0, i, 0)),
            pl.BlockSpec((F, F), lambda i: (0, 0)),
            pl.BlockSpec((1, F), lambda i: (0, 0)),
        ],
        out_specs=[
            pl.BlockSpec((NC, BM, FH), lambda i: (0, i, 0)),
            pl.BlockSpec((BM, FH), lambda i: (i, 0)),
            pl.BlockSpec((NC, BM, FH), lambda i: (0, i, 0)),
        ],
        out_shape=[
            jax.ShapeDtypeStruct((NC, NPAD, FH), jnp.float32),
            jax.ShapeDtypeStruct((NPAD, FH), jnp.float32),
            jax.ShapeDtypeStruct((NC, NPAD, FH), jnp.float32),
        ],
    )(x, m, degp, w, b)


def _score_body(h_ref, nei_ref, degp_ref, m_ref, s_ref):
    dp = degp_ref[...]
    deg = dp[0, :, 0:1] + dp[1, :, 0:1]
    di = jnp.maximum(deg, 1.0)
    hp = h_ref[...]
    hfull = jnp.concatenate([hp[0], hp[1]], axis=1)
    np_ = nei_ref[...]
    nei = jnp.concatenate([np_[0], np_[1]], axis=1) / di
    s = jnp.sum(jnp.abs(hfull - nei), axis=1, keepdims=True)
    s_ref[...] = jnp.where(m_ref[...] > 0.5, s, -1.0)


def _tc_score(h2, nei, degp, m):
    return pl.pallas_call(
        _score_body,
        grid=(NPAD // BM,),
        in_specs=[
            pl.BlockSpec((NC, BM, FH), lambda i: (0, i, 0)),
            pl.BlockSpec((NC, BM, FH), lambda i: (0, i, 0)),
            pl.BlockSpec((NC, BM, FH), lambda i: (0, i, 0)),
            pl.BlockSpec((BM, 1), lambda i: (i, 0)),
        ],
        out_specs=pl.BlockSpec((BM, 1), lambda i: (i, 0)),
        out_shape=jax.ShapeDtypeStruct((NPAD, 1), jnp.float32),
    )(h2, nei, degp, m)


def _topk_body(k, srow_ref, scol_ref, mask_ref):
    rl = lax.broadcasted_iota(jnp.int32, (NPERP, NPERP), 0)
    cl = lax.broadcasted_iota(jnp.int32, (NPERP, NPERP), 1)
    lt = (rl < cl)
    for g in range(B):
        si = srow_ref[pl.ds(g, 1), :]
        sj = scol_ref[g, :, :]
        gt = (sj > si).astype(jnp.float32)
        tie = ((sj == si) & lt).astype(jnp.float32)
        rank = jnp.sum(gt + tie, axis=0, keepdims=True)
        mask_ref[pl.ds(g, 1), :] = (rank < float(k)).astype(jnp.float32)


def _tc_topk(sg, sg3, k):
    return pl.pallas_call(
        functools.partial(_topk_body, k),
        out_shape=jax.ShapeDtypeStruct((B, NPERP), jnp.float32),
    )(sg, sg3)


def _head_body(h0_ref, h1_ref, m_ref, w1_ref, b1_ref, w2_ref, b2_ref,
               w3_ref, b3_ref, out_ref):
    h = jnp.concatenate([h0_ref[...], h1_ref[...]], axis=2)
    m = m_ref[...]
    mx = jnp.max(jnp.where(m > 0.5, h, -1e30), axis=1)
    sm = jnp.sum(h * m, axis=1) / float(K2)
    x3 = jnp.concatenate([mx, sm], axis=1)
    z = jnp.maximum(jnp.dot(x3, w1_ref[...],
                            preferred_element_type=jnp.float32)
                    + b1_ref[...], 0.0)
    z = jnp.maximum(jnp.dot(z, w2_ref[...],
                            preferred_element_type=jnp.float32)
                    + b2_ref[...], 0.0)
    o = jnp.dot(z, w3_ref[...], preferred_element_type=jnp.float32) \
        + b3_ref[...]
    omax = jnp.max(o, axis=1, keepdims=True)
    osh = o - omax
    lse = jnp.log(jnp.sum(jnp.exp(osh), axis=1, keepdims=True))
    out_ref[...] = osh - lse


def _tc_head(h0g, h1g, mg, w1, b1, w2, b2, w3, b3):
    return pl.pallas_call(
        _head_body,
        out_shape=jax.ShapeDtypeStruct((B, 10), jnp.float32),
    )(h0g, h1g, mg, w1, b1, w2, b2, w3, b3)


def kernel(x, edge_index, batch, skew, W1, b1, Wc, bc, W2, b2, W3, b3,
           Wl1, bl1, Wl2, bl2, Wl3, bl3):
    f32 = jnp.float32
    xp = jnp.pad(x, ((0, NPAD - N), (0, 0)))
    pad_e = jnp.full((EPAD - E,), ZROW, jnp.int32)
    src3 = jnp.concatenate([edge_index[0], pad_e]).reshape(NTILES, NSTEP, CH)
    dst3 = jnp.concatenate([edge_index[1], pad_e]).reshape(NTILES, NSTEP, CH)
    ones_cd = jnp.ones((CH, FH), f32)
    zhalf = jnp.zeros((NPAD, FH), f32)
    m = jnp.pad(jnp.ones((N,), f32), (0, NPAD - N))

    h = xp
    layers = [(W1, b1), (W2, b2), (W3, b3)]
    for l in range(3):
        W, bvec = layers[l]
        b2d = bvec.reshape(1, F)
        m2d = m.reshape(NPAD, 1)
        idxg, idxs, degp = _maskedge(m, src3, dst3, ones_cd, zhalf)
        tab, dinvb, selfh = _tc_mm(h, m2d, degp, W, b2d, half_in=(l > 0))
        if l < 2:
            h, nei = _mega(tab, idxg, idxs, dinvb, selfh, zhalf, info=True)
            s = _tc_score(h, nei, degp, m2d)
            sg = jnp.pad(s[:N, 0].reshape(B, NPER), ((0, 0), (0, NPERP - NPER)),
                         constant_values=-1.0)
            mask = _tc_topk(sg, sg.reshape(B, NPERP, 1), K1 if l == 0 else K2)
            m = jnp.pad(mask[:, :NPER].reshape(-1), (0, NPAD - N))
        else:
            (h,) = _mega(tab, idxg, idxs, dinvb, selfh, zhalf, info=False)

    h0g = jnp.pad(h[0][:N].reshape(B, NPER, FH),
                  ((0, 0), (0, NPERP - NPER), (0, 0)))
    h1g = jnp.pad(h[1][:N].reshape(B, NPER, FH),
                  ((0, 0), (0, NPERP - NPER), (0, 0)))
    mg = jnp.pad(m[:N].reshape(B, NPER, 1), ((0, 0), (0, NPERP - NPER), (0, 0)))
    return _tc_head(h0g, h1g, mg, Wl1, bl1.reshape(1, F), Wl2,
                    bl2.reshape(1, 64), Wl3, bl3.reshape(1, 10))


# speedup vs baseline: 1.0581x; 1.0581x over previous
import functools

import jax
import jax.numpy as jnp
from jax import lax
from jax.experimental import pallas as pl
from jax.experimental.pallas import tpu as pltpu
from jax.experimental.pallas import tpu_sc as plsc

N = 10000
E = 320000
B = 8
NPER = 1250
F = 128
K1 = 625
K2 = 313
NPAD = 10240
ZROW = N
NC = 2
NS = 16
NTILES = NC * NS
CH = 128
NSTEP = 79
EPT = NSTEP * CH
EPAD = NTILES * EPT
RPS = NPAD // NS
FH = 64
KB = 5
NSTEP2 = 160
EPAD2 = NS * NSTEP2 * CH
NPERP = 1280

@functools.cache
def _mesh():
    return plsc.VectorSubcoreMesh(
        core_axis_name="c", subcore_axis_name="s",
        num_cores=NC, num_subcores=NS)


def _maskedge_body(m_hbm, src_hbm, dst_hbm, ones_hbm, zdeg_hbm,
                   idxg_hbm, idxs_hbm, degp_hbm,
                   m_v, s_v, d_v, og_v, os_v, isb_v, ones_v, deg_sp):
    c = lax.axis_index("c")
    s = lax.axis_index("s")
    wid = c * NS + s
    pltpu.sync_copy(m_hbm, m_v)
    pltpu.sync_copy(src_hbm.at[wid], s_v)
    pltpu.sync_copy(dst_hbm.at[wid], d_v)
    pltpu.sync_copy(ones_hbm, ones_v)
    pltpu.sync_copy(zdeg_hbm.at[pl.ds(s * RPS, RPS)],
                    deg_sp.at[pl.ds(s * RPS, RPS)])
    plsc.subcore_barrier()

    def step(j, carry):
        for k in range(CH // 16):
            s16 = s_v[j, pl.ds(k * 16, 16)]
            d16 = d_v[j, pl.ds(k * 16, 16)]
            mg = plsc.load_gather(m_v, [s16])
            md = plsc.load_gather(m_v, [d16])
            keep = (mg * md) > 0.5
            trash16 = lax.iota(jnp.int32, 16) + (ZROW + 16 * k)
            os16 = jnp.where(keep, d16, trash16)
            og_v[j, pl.ds(k * 16, 16)] = jnp.where(keep, s16, trash16)
            os_v[j, pl.ds(k * 16, 16)] = os16
            isb_v[pl.ds(k * 16, 16)] = os16
        pltpu.sync_copy(ones_v, deg_sp.at[isb_v], add=True)
        return carry

    lax.fori_loop(0, NSTEP, step, 0)
    pltpu.sync_copy(og_v, idxg_hbm.at[wid])
    pltpu.sync_copy(os_v, idxs_hbm.at[wid])
    plsc.subcore_barrier()
    pltpu.sync_copy(deg_sp.at[pl.ds(s * RPS, RPS)],
                    degp_hbm.at[c, pl.ds(s * RPS, RPS)])


@functools.cache
def _maskedge_kernel():
    return pl.kernel(
        _maskedge_body,
        out_type=(
            jax.ShapeDtypeStruct((NTILES, NSTEP, CH), jnp.int32),
            jax.ShapeDtypeStruct((NTILES, NSTEP, CH), jnp.int32),
            jax.ShapeDtypeStruct((NC, NPAD, FH), jnp.float32),
        ),
        mesh=_mesh(),
        compiler_params=pltpu.CompilerParams(
            needs_layout_passes=False, use_tc_tiling_on_sc=False),
        scratch_types=[
            pltpu.VMEM((NPAD,), jnp.float32),
            pltpu.VMEM((NSTEP, CH), jnp.int32),
            pltpu.VMEM((NSTEP, CH), jnp.int32),
            pltpu.VMEM((NSTEP, CH), jnp.int32),
            pltpu.VMEM((NSTEP, CH), jnp.int32),
            pltpu.VMEM((CH,), jnp.int32),
            pltpu.VMEM((CH, FH), jnp.float32),
            pltpu.VMEM_SHARED((NPAD, FH), jnp.float32),
        ],
    )


def _maskedge(m, src3, dst3, ones_cd, zdeg):
    return _maskedge_kernel()(m, src3, dst3, ones_cd, zdeg)


def _scatter_body(table_hbm, idxg_hbm, idxs_hbm, zrows_hbm, out_hbm,
                  ig_v, is_v, igb_v, isb_v, rows_v, acc_sp, gsem, ssem):
    c = lax.axis_index("c")
    s = lax.axis_index("s")
    pltpu.sync_copy(idxg_hbm.at[s], ig_v)
    pltpu.sync_copy(idxs_hbm.at[s], is_v)
    pltpu.sync_copy(zrows_hbm.at[pl.ds(s * RPS, RPS)],
                    acc_sp.at[pl.ds(s * RPS, RPS)])
    plsc.subcore_barrier()

    def step(g, carry):
        base = g * KB
        gd = []
        for b in range(KB):
            j = base + b
            for k in range(CH // 16):
                igb_v[b][pl.ds(k * 16, 16)] = ig_v[j, pl.ds(k * 16, 16)]
                isb_v[b][pl.ds(k * 16, 16)] = is_v[j, pl.ds(k * 16, 16)]
            gd.append(pltpu.async_copy(table_hbm.at[c].at[igb_v[b]],
                                       rows_v[b], gsem))
        sd = []
        for b in range(KB):
            gd[b].wait()
            sd.append(pltpu.async_copy(rows_v[b], acc_sp.at[isb_v[b]],
                                       ssem, add=True))
        for b in range(KB):
            sd[b].wait()
        return carry

    lax.fori_loop(0, NSTEP2 // KB, step, 0)
    plsc.subcore_barrier()
    pltpu.sync_copy(acc_sp.at[pl.ds(s * RPS, RPS)],
                    out_hbm.at[c, pl.ds(s * RPS, RPS)])


@functools.cache
def _scatter_kernel():
    return pl.kernel(
        _scatter_body,
        out_type=jax.ShapeDtypeStruct((NC, NPAD, FH), jnp.float32),
        mesh=_mesh(),
        compiler_params=pltpu.CompilerParams(
            needs_layout_passes=False, use_tc_tiling_on_sc=False),
        scratch_types=[
            pltpu.VMEM((NSTEP2, CH), jnp.int32),
            pltpu.VMEM((NSTEP2, CH), jnp.int32),
            [pltpu.VMEM((CH,), jnp.int32) for _ in range(KB)],
            [pltpu.VMEM((CH,), jnp.int32) for _ in range(KB)],
            [pltpu.VMEM((CH, FH), jnp.float32) for _ in range(KB)],
            pltpu.VMEM_SHARED((NPAD, FH), jnp.float32),
            pltpu.SemaphoreType.DMA,
            pltpu.SemaphoreType.DMA,
        ],
    )


def _scatter(table2, idxg, idxs, zhalf):
    def pad2(a):
        flat = a.reshape(-1)
        spread = ZROW + jnp.arange(EPAD2 - EPAD, dtype=jnp.int32) % 128
        flat = jnp.concatenate([flat, spread])
        return flat.reshape(NS, NSTEP2, CH)
    return _scatter_kernel()(table2, pad2(idxg), pad2(idxs), zhalf)


BM = 1024


def _mm_body(x_ref, m_ref, degp_ref, w_ref, h_ref, hs_ref, dinv_ref):
    dp = degp_ref[...]
    deg = dp[0, :, 0:1] + dp[1, :, 0:1]
    dinv = lax.rsqrt(deg + 1.0)
    xm = x_ref[...] * m_ref[...]
    h = jnp.dot(xm, w_ref[...], preferred_element_type=jnp.float32)
    hs = h * dinv
    h_ref[...] = h
    hs_ref[0, :, :] = hs[:, 0:FH]
    hs_ref[1, :, :] = hs[:, FH:F]
    dinv_ref[...] = dinv


def _tc_mm(x, m, degp, w):
    return pl.pallas_call(
        _mm_body,
        grid=(NPAD // BM,),
        in_specs=[
            pl.BlockSpec((BM, F), lambda i: (i, 0)),
            pl.BlockSpec((BM, 1), lambda i: (i, 0)),
            pl.BlockSpec((NC, BM, FH), lambda i: (0, i, 0)),
            pl.BlockSpec((F, F), lambda i: (0, 0)),
        ],
        out_specs=[
            pl.BlockSpec((BM, F), lambda i: (i, 0)),
            pl.BlockSpec((NC, BM, FH), lambda i: (0, i, 0)),
            pl.BlockSpec((BM, 1), lambda i: (i, 0)),
        ],
        out_shape=[
            jax.ShapeDtypeStruct((NPAD, F), jnp.float32),
            jax.ShapeDtypeStruct((NC, NPAD, FH), jnp.float32),
            jax.ShapeDtypeStruct((NPAD, 1), jnp.float32),
        ],
    )(x, m, degp, w)


def _post_body(s_ref, h_ref, dinv_ref, b_ref, out_ref, hh_ref):
    i = pl.program_id(0)
    sp = s_ref[...]
    sfull = jnp.concatenate([sp[0], sp[1]], axis=1)
    dinv = dinv_ref[...]
    h = h_ref[...]
    agg = dinv * sfull + dinv * dinv * h + b_ref[...]
    rows = i * BM + lax.broadcasted_iota(jnp.int32, (BM, 1), 0)
    valid = (rows < N).astype(jnp.float32)
    out = jnp.maximum(agg, 0.0) * valid
    out_ref[...] = out
    hh_ref[0, :, :] = out[:, 0:FH]
    hh_ref[1, :, :] = out[:, FH:F]


def _tc_post(s2, h, dinv, b):
    return pl.pallas_call(
        _post_body,
        grid=(NPAD // BM,),
        in_specs=[
            pl.BlockSpec((NC, BM, FH), lambda i: (0, i, 0)),
            pl.BlockSpec((BM, F), lambda i: (i, 0)),
            pl.BlockSpec((BM, 1), lambda i: (i, 0)),
            pl.BlockSpec((1, F), lambda i: (0, 0)),
        ],
        out_specs=[
            pl.BlockSpec((BM, F), lambda i: (i, 0)),
            pl.BlockSpec((NC, BM, FH), lambda i: (0, i, 0)),
        ],
        out_shape=[
            jax.ShapeDtypeStruct((NPAD, F), jnp.float32),
            jax.ShapeDtypeStruct((NC, NPAD, FH), jnp.float32),
        ],
    )(s2, h, dinv, b)


def _score_body(h_ref, nei_ref, degp_ref, m_ref, s_ref):
    dp = degp_ref[...]
    deg = dp[0, :, 0:1] + dp[1, :, 0:1]
    di = jnp.maximum(deg, 1.0)
    np_ = nei_ref[...]
    nei = jnp.concatenate([np_[0], np_[1]], axis=1) / di
    s = jnp.sum(jnp.abs(h_ref[...] - nei), axis=1, keepdims=True)
    s_ref[...] = jnp.where(m_ref[...] > 0.5, s, -1.0)


def _tc_score(h, nei, degp, m):
    return pl.pallas_call(
        _score_body,
        grid=(NPAD // BM,),
        in_specs=[
            pl.BlockSpec((BM, F), lambda i: (i, 0)),
            pl.BlockSpec((NC, BM, FH), lambda i: (0, i, 0)),
            pl.BlockSpec((NC, BM, FH), lambda i: (0, i, 0)),
            pl.BlockSpec((BM, 1), lambda i: (i, 0)),
        ],
        out_specs=pl.BlockSpec((BM, 1), lambda i: (i, 0)),
        out_shape=jax.ShapeDtypeStruct((NPAD, 1), jnp.float32),
    )(h, nei, degp, m)


def _topk_body(k, srow_ref, scol_ref, mask_ref):
    rl = lax.broadcasted_iota(jnp.int32, (NPERP, NPERP), 0)
    cl = lax.broadcasted_iota(jnp.int32, (NPERP, NPERP), 1)
    lt = (rl < cl)
    for g in range(B):
        si = srow_ref[pl.ds(g, 1), :]
        sj = scol_ref[g, :, :]
        gt = (sj > si).astype(jnp.float32)
        tie = ((sj == si) & lt).astype(jnp.float32)
        rank = jnp.sum(gt + tie, axis=0, keepdims=True)
        mask_ref[pl.ds(g, 1), :] = (rank < float(k)).astype(jnp.float32)


def _tc_topk(sg, sg3, k):
    return pl.pallas_call(
        functools.partial(_topk_body, k),
        out_shape=jax.ShapeDtypeStruct((B, NPERP), jnp.float32),
    )(sg, sg3)


def _head_body(h_ref, m_ref, w1_ref, b1_ref, w2_ref, b2_ref, w3_ref, b3_ref,
               out_ref):
    h = h_ref[...]
    m = m_ref[...]
    mx = jnp.max(jnp.where(m > 0.5, h, -1e30), axis=1)
    sm = jnp.sum(h * m, axis=1) / float(K2)
    x3 = jnp.concatenate([mx, sm], axis=1)
    z = jnp.maximum(jnp.dot(x3, w1_ref[...],
                            preferred_element_type=jnp.float32)
                    + b1_ref[...], 0.0)
    z = jnp.maximum(jnp.dot(z, w2_ref[...],
                            preferred_element_type=jnp.float32)
                    + b2_ref[...], 0.0)
    o = jnp.dot(z, w3_ref[...], preferred_element_type=jnp.float32) \
        + b3_ref[...]
    omax = jnp.max(o, axis=1, keepdims=True)
    osh = o - omax
    lse = jnp.log(jnp.sum(jnp.exp(osh), axis=1, keepdims=True))
    out_ref[...] = osh - lse


def _tc_head(hg, mg, w1, b1, w2, b2, w3, b3):
    return pl.pallas_call(
        _head_body,
        out_shape=jax.ShapeDtypeStruct((B, 10), jnp.float32),
    )(hg, mg, w1, b1, w2, b2, w3, b3)


def kernel(x, edge_index, batch, skew, W1, b1, Wc, bc, W2, b2, W3, b3,
           Wl1, bl1, Wl2, bl2, Wl3, bl3):
    f32 = jnp.float32
    xp = jnp.pad(x, ((0, NPAD - N), (0, 0)))
    pad_e = jnp.full((EPAD - E,), ZROW, jnp.int32)
    src3 = jnp.concatenate([edge_index[0], pad_e]).reshape(NTILES, NSTEP, CH)
    dst3 = jnp.concatenate([edge_index[1], pad_e]).reshape(NTILES, NSTEP, CH)
    ones_cd = jnp.ones((CH, FH), f32)
    zhalf = jnp.zeros((NPAD, FH), f32)
    m = jnp.pad(jnp.ones((N,), f32), (0, NPAD - N))

    h = xp
    layers = [(W1, b1), (W2, b2), (W3, b3)]
    for l in range(3):
        W, bvec = layers[l]
        b2d = bvec.reshape(1, F)
        m2d = m.reshape(NPAD, 1)
        idxg, idxs, degp = _maskedge(m, src3, dst3, ones_cd, zhalf)
        H, Hs2, dinv = _tc_mm(h, m2d, degp, W)
        S = _scatter(Hs2, idxg, idxs, zhalf)
        h, hh = _tc_post(S, H, dinv, b2d)
        if l < 2:
            NEI = _scatter(hh, idxg, idxs, zhalf)
            s = _tc_score(h, NEI, degp, m2d)
            sg = jnp.pad(s[:N, 0].reshape(B, NPER), ((0, 0), (0, NPERP - NPER)),
                         constant_values=-1.0)
            mask = _tc_topk(sg, sg.reshape(B, NPERP, 1), K1 if l == 0 else K2)
            m = jnp.pad(mask[:, :NPER].reshape(-1), (0, NPAD - N))

    hg = jnp.pad(h[:N].reshape(B, NPER, F), ((0, 0), (0, NPERP - NPER), (0, 0)))
    mg = jnp.pad(m[:N].reshape(B, NPER, 1), ((0, 0), (0, NPERP - NPER), (0, 0)))
    return _tc_head(hg, mg, Wl1, bl1.reshape(1, F), Wl2, bl2.reshape(1, 64),
                    Wl3, bl3.reshape(1, 10))
